# baseline (speedup 1.0000x reference)
import functools

import jax
import jax.numpy as jnp
from jax import lax
from jax.experimental import pallas as pl
from jax.experimental.pallas import tpu as pltpu
from jax.experimental.pallas import tpu_sc as plsc

B = 16384
F = 26
E = 16
N = B * F
NW = 32
PW = N // NW
CH = 1664
NCH = PW // CH
D_HIDDEN = F * E
VROWS = 100000


def _sc_gather(x_flat, off_flat, embed_table, lin_flat):
    mesh = plsc.VectorSubcoreMesh(core_axis_name="c", subcore_axis_name="s")

    @functools.partial(
        pl.kernel,
        out_type=(
            jax.ShapeDtypeStruct((N, E), jnp.float32),
            jax.ShapeDtypeStruct((N,), jnp.float32),
        ),
        mesh=mesh,
        compiler_params=pltpu.CompilerParams(use_tc_tiling_on_sc=False),
        scratch_types=[
            pltpu.VMEM((PW,), jnp.int32),
            pltpu.VMEM((PW,), jnp.int32),
            pltpu.VMEM((2, CH, E), jnp.float32),
            pltpu.VMEM((2, CH), jnp.float32),
            pltpu.SemaphoreType.DMA,
            pltpu.SemaphoreType.DMA,
        ],
    )
    def k(x_hbm, off_hbm, emb_hbm, lin_hbm, out_e, out_l,
          idx_v, off_v, ebuf, lbuf, esem, lsem):
        wid = lax.axis_index("s") * 2 + lax.axis_index("c")
        base = wid * PW

        pltpu.sync_copy(x_hbm.at[pl.ds(base, PW)], idx_v)
        pltpu.sync_copy(off_hbm, off_v)

        def add_off(j, _):
            for u in range(8):
                sl = pl.ds(j * 128 + u * 16, 16)
                idx_v[sl] = idx_v[sl] + off_v[sl]
            return 0

        lax.fori_loop(0, PW // 128, add_off, 0)

        ed = [None, None]
        ld = [None, None]
        for j in range(NCH):
            b = j & 1
            isl = idx_v.at[pl.ds(j * CH, CH)]
            ed[b] = pltpu.async_copy(emb_hbm.at[isl], ebuf.at[b], esem)
            ld[b] = pltpu.async_copy(lin_hbm.at[isl], lbuf.at[b], lsem)
            if j > 0:
                p = (j - 1) & 1
                ed[p].wait()
                pltpu.sync_copy(ebuf.at[p],
                                out_e.at[pl.ds(base + (j - 1) * CH, CH)])
                ld[p].wait()
                pltpu.sync_copy(lbuf.at[p],
                                out_l.at[pl.ds(base + (j - 1) * CH, CH)])
        p = (NCH - 1) & 1
        ed[p].wait()
        pltpu.sync_copy(ebuf.at[p], out_e.at[pl.ds(base + (NCH - 1) * CH, CH)])
        ld[p].wait()
        pltpu.sync_copy(lbuf.at[p], out_l.at[pl.ds(base + (NCH - 1) * CH, CH)])

    return k(x_flat, off_flat, embed_table, lin_flat)


def _tc_mlp(h, linv, W1, b1, W2, b2, w3row, b3):
    BM = 1024
    grid = (B // BM,)

    def body(h_ref, l_ref, w1_ref, b1_ref, w2_ref, b2_ref, w3_ref, b3_ref,
             o_ref):
        hb = h_ref[...]
        a1 = jnp.dot(hb, w1_ref[...], preferred_element_type=jnp.float32)
        a1 = jnp.maximum(a1 + b1_ref[...], 0.0)
        a2 = jnp.dot(a1, w2_ref[...], preferred_element_type=jnp.float32)
        a2 = jnp.maximum(a2 + b2_ref[...], 0.0)
        deep = jnp.sum(a2 * w3_ref[...], axis=1) + b3_ref[0, 0]
        lin_b = jnp.sum(l_ref[...], axis=1)
        o_ref[...] = (deep + lin_b).reshape(BM // 128, 128)

    out = pl.pallas_call(
        body,
        grid=grid,
        in_specs=[
            pl.BlockSpec((BM, D_HIDDEN), lambda i: (i, 0)),
            pl.BlockSpec((BM, F), lambda i: (i, 0)),
            pl.BlockSpec((D_HIDDEN, 256), lambda i: (0, 0)),
            pl.BlockSpec((1, 256), lambda i: (0, 0)),
            pl.BlockSpec((256, 128), lambda i: (0, 0)),
            pl.BlockSpec((1, 128), lambda i: (0, 0)),
            pl.BlockSpec((1, 128), lambda i: (0, 0)),
            pl.BlockSpec((1, 1), lambda i: (0, 0)),
        ],
        out_specs=pl.BlockSpec((BM // 128, 128), lambda i: (i, 0)),
        out_shape=jax.ShapeDtypeStruct((B // 128, 128), jnp.float32),
    )(h, linv, W1, b1, W2, b2, w3row, b3)
    return out.reshape(B)


def kernel(x, embed_table, lin_table, W1, b1, W2, b2, W3, b3):
    x_flat = x.reshape(N)
    off_flat = (jnp.arange(PW, dtype=jnp.int32) % F) * VROWS
    lin_flat = lin_table.reshape(lin_table.shape[0])
    emb_rows, lin_rows = _sc_gather(x_flat, off_flat, embed_table, lin_flat)
    h = emb_rows.reshape(B, D_HIDDEN)
    linv = lin_rows.reshape(B, F)
    return _tc_mlp(h, linv, W1, b1.reshape(1, 256), W2, b2.reshape(1, 128),
                   W3.reshape(1, 128), b3.reshape(1, 1))
